# baseline (speedup 1.0000x reference)
import functools

import jax
import jax.numpy as jnp
from jax import lax
from jax.experimental import pallas as pl
from jax.experimental.pallas import tpu as pltpu
from jax.experimental.pallas import tpu_sc as plsc

B = 16384
EMB = 32
OUT_D = EMB + 1
NC, NS, L = 2, 16, 16
NW = NC * NS
BW = B // NW
CHUNK = 128
NCH = BW // CHUNK
NSLOT = 2

_mesh = plsc.VectorSubcoreMesh(core_axis_name="c", subcore_axis_name="s")


def _gather_body(uids, iids, ut, it, urows_hbm, irows_hbm,
                 uidx, iidx, urows, irows, usems, isems):
  wid = lax.axis_index("s") * NC + lax.axis_index("c")
  base = wid * BW

  pltpu.sync_copy(uids.at[pl.ds(base, BW)], uidx)
  pltpu.sync_copy(iids.at[pl.ds(base, BW)], iidx)

  def fire(tbl, idx, rows, sems, j):
    return pltpu.async_copy(
        tbl.at[idx.at[pl.ds(j * CHUNK, CHUNK)]],
        rows.at[j % NSLOT], sems.at[j % NSLOT])

  ucopies = [fire(ut, uidx, urows, usems, j) for j in range(NSLOT)]
  icopies = [fire(it, iidx, irows, isems, j) for j in range(NSLOT)]

  for j in range(NCH):
    ucopies[j].wait()
    pltpu.sync_copy(urows.at[j % NSLOT],
                    urows_hbm.at[pl.ds(base + j * CHUNK, CHUNK)])
    if j + NSLOT < NCH:
      ucopies.append(fire(ut, uidx, urows, usems, j + NSLOT))

  for j in range(NCH):
    icopies[j].wait()
    pltpu.sync_copy(irows.at[j % NSLOT],
                    irows_hbm.at[pl.ds(base + j * CHUNK, CHUNK)])
    if j + NSLOT < NCH:
      icopies.append(fire(it, iidx, irows, isems, j + NSLOT))


_gather_call = functools.partial(
    pl.kernel,
    out_type=[
        jax.ShapeDtypeStruct((B, EMB), jnp.float32),
        jax.ShapeDtypeStruct((B, EMB), jnp.float32),
    ],
    mesh=_mesh,
    scratch_types=[
        pltpu.VMEM((BW,), jnp.int32),
        pltpu.VMEM((BW,), jnp.int32),
        pltpu.VMEM((NSLOT, CHUNK, EMB), jnp.float32),
        pltpu.VMEM((NSLOT, CHUNK, EMB), jnp.float32),
        pltpu.SemaphoreType.DMA((NSLOT,)),
        pltpu.SemaphoreType.DMA((NSLOT,)),
    ],
    compiler_params=pltpu.CompilerParams(use_tc_tiling_on_sc=False,
                                         needs_layout_passes=False),
)(_gather_body)


def _assemble_chunk(idx_ref, rbuf, feat_ref, j):
  lanes = lax.iota(jnp.int32, L)
  zeros = jnp.zeros((L,), jnp.int32)

  def group(g, _):
    sv = j * CHUNK + g * L + lanes
    lid = g * L + lanes
    ids = plsc.load_gather(idx_ref, [sv])
    plsc.store_scatter(feat_ref, [zeros, sv], ids.astype(jnp.float32))
    for r in range(EMB):
      vals = plsc.load_gather(rbuf, [lid, zeros + r])
      plsc.store_scatter(feat_ref, [zeros + (1 + r), sv], vals)
    return 0

  lax.fori_loop(0, CHUNK // L, group, 0, unroll=False)


def _assemble_body(uids, iids, urows_hbm, irows_hbm, uoutT, ioutT,
                   uidx, iidx, ubuf, ibuf, ufeat, ifeat, usems, isems):
  wid = lax.axis_index("s") * NC + lax.axis_index("c")
  base = wid * BW

  pltpu.sync_copy(uids.at[pl.ds(base, BW)], uidx)
  pltpu.sync_copy(iids.at[pl.ds(base, BW)], iidx)

  def fire(src, buf, sems, j):
    return pltpu.async_copy(src.at[pl.ds(base + j * CHUNK, CHUNK)],
                            buf.at[j % NSLOT], sems.at[j % NSLOT])

  ucopies = [fire(urows_hbm, ubuf, usems, j) for j in range(NSLOT)]
  icopies = [fire(irows_hbm, ibuf, isems, j) for j in range(NSLOT)]

  for j in range(NCH):
    ucopies[j].wait()
    _assemble_chunk(uidx, ubuf.at[j % NSLOT], ufeat, j)
    if j + NSLOT < NCH:
      ucopies.append(fire(urows_hbm, ubuf, usems, j + NSLOT))
  pltpu.sync_copy(ufeat, uoutT.at[:, pl.ds(base, BW)])

  for j in range(NCH):
    icopies[j].wait()
    _assemble_chunk(iidx, ibuf.at[j % NSLOT], ifeat, j)
    if j + NSLOT < NCH:
      icopies.append(fire(irows_hbm, ibuf, isems, j + NSLOT))
  pltpu.sync_copy(ifeat, ioutT.at[:, pl.ds(base, BW)])


_assemble_call = functools.partial(
    pl.kernel,
    out_type=[
        jax.ShapeDtypeStruct((OUT_D, B), jnp.float32),
        jax.ShapeDtypeStruct((OUT_D, B), jnp.float32),
    ],
    mesh=_mesh,
    scratch_types=[
        pltpu.VMEM((BW,), jnp.int32),
        pltpu.VMEM((BW,), jnp.int32),
        pltpu.VMEM((NSLOT, CHUNK, EMB), jnp.float32),
        pltpu.VMEM((NSLOT, CHUNK, EMB), jnp.float32),
        pltpu.VMEM((OUT_D, BW), jnp.float32),
        pltpu.VMEM((OUT_D, BW), jnp.float32),
        pltpu.SemaphoreType.DMA((NSLOT,)),
        pltpu.SemaphoreType.DMA((NSLOT,)),
    ],
    compiler_params=pltpu.CompilerParams(needs_layout_passes=False),
)(_assemble_body)


@jax.jit
def kernel(user_id, item_id, user_table, item_table):
  uids = user_id.reshape(B).astype(jnp.int32)
  iids = item_id.reshape(B).astype(jnp.int32)
  urows, irows = _gather_call(uids, iids, user_table, item_table)
  uT, iT = _assemble_call(uids, iids, urows, irows)
  return uT.T, iT.T
